# baseline (speedup 1.0000x reference)
import functools

import jax
import jax.numpy as jnp
from jax import lax
from jax.experimental import pallas as pl
from jax.experimental.pallas import tpu as pltpu
from jax.experimental.pallas import tpu_sc as plsc

N = 10000
E = 320000
B = 100000
H = 128
PROX_W = 0.3

NC, NS = 2, 16
NW = NC * NS

E_PER_TILE = E // NW
EC = 80
NCH_E = E_PER_TILE // EC

BPAD = 102400
BC = 80
NCH0 = 62
NCH1 = 18
B_PER_TILE = BPAD // NW

N_PAD = 10240
NP_TILE = N_PAD // NS

_sc_built = {}


def _sc_mesh():
    return plsc.VectorSubcoreMesh(
        core_axis_name="c", subcore_axis_name="s", num_cores=NC, num_subcores=NS
    )


def _sc_deg_body(dst_hbm, out_hbm, dst_v, cnt_v):
    c = lax.axis_index("c")
    s = lax.axis_index("s")
    wid = c * NS + s
    pltpu.sync_copy(dst_hbm.at[pl.ds(wid * E_PER_TILE, E_PER_TILE)], dst_v)

    def zbody(i, carry):
        cnt_v[pl.ds(i * 16, 16)] = jnp.zeros((16,), jnp.float32)
        return carry

    lax.fori_loop(0, N_PAD // 16, zbody, 0)
    ones16 = jnp.full((16,), 1.0, jnp.float32)

    def body(i, carry):
        idx = dst_v[pl.ds(i * 16, 16)]
        plsc.addupdate_scatter(cnt_v, [idx], ones16)
        return carry

    lax.fori_loop(0, E_PER_TILE // 16, body, 0)
    pltpu.sync_copy(cnt_v, out_hbm.at[pl.ds(wid * N_PAD, N_PAD)])


def _sc_deg(dst_flat):
    fn = _sc_built.get("deg")
    if fn is None:
        fn = pl.kernel(
            _sc_deg_body,
            out_type=jax.ShapeDtypeStruct((NW * N_PAD,), jnp.float32),
            mesh=_sc_mesh(),
            scratch_types=[
                pltpu.VMEM((E_PER_TILE,), jnp.int32),
                pltpu.VMEM((N_PAD,), jnp.float32),
            ],
            compiler_params=pltpu.CompilerParams(use_tc_tiling_on_sc=False, needs_layout_passes=False),
        )
        _sc_built["deg"] = fn
    return fn(dst_flat)


def _sc_agg_body(g_hbm, src_hbm, dst_hbm, z2_hbm, out_hbm, src_v, dst_v,
                 rows_a, rows_b, acc, sem_a, sem_b):
    c = lax.axis_index("c")
    s = lax.axis_index("s")
    wid = c * NS + s
    pltpu.sync_copy(src_hbm.at[wid], src_v)
    pltpu.sync_copy(dst_hbm.at[wid], dst_v)
    pltpu.sync_copy(z2_hbm, acc.at[pl.ds(s * NP_TILE, NP_TILE)])
    plsc.subcore_barrier()

    def start(j, buf, sem):
        return pltpu.async_copy(g_hbm.at[src_v.at[j]], buf, sem)

    def wait_sc(j, buf, sem):
        pltpu.make_async_copy(g_hbm.at[src_v.at[j]], buf, sem).wait()
        pltpu.sync_copy(buf, acc.at[dst_v.at[j]], add=True)

    start(0, rows_a, sem_a)

    def body(g, carry):
        c0 = 2 * g
        start(c0 + 1, rows_b, sem_b)
        wait_sc(c0, rows_a, sem_a)
        start(c0 + 2, rows_a, sem_a)
        wait_sc(c0 + 1, rows_b, sem_b)
        return carry

    lax.fori_loop(0, NCH_E // 2, body, 0)
    wait_sc(NCH_E - 1, rows_a, sem_a)
    plsc.subcore_barrier()
    pltpu.sync_copy(
        acc.at[pl.ds(s * NP_TILE, NP_TILE)],
        out_hbm.at[c, pl.ds(s * NP_TILE, NP_TILE)],
    )


def _sc_agg(g, src, dst, z2):
    fn = _sc_built.get("agg")
    if fn is None:
        fn = pl.kernel(
            _sc_agg_body,
            out_type=jax.ShapeDtypeStruct((NC, N_PAD, H), jnp.float32),
            mesh=_sc_mesh(),
            scratch_types=[
                pltpu.VMEM((NCH_E, EC), jnp.int32),
                pltpu.VMEM((NCH_E, EC), jnp.int32),
                pltpu.VMEM((EC, H), jnp.float32),
                pltpu.VMEM((EC, H), jnp.float32),
                pltpu.VMEM_SHARED((N_PAD, H), jnp.float32),
                pltpu.SemaphoreType.DMA,
                pltpu.SemaphoreType.DMA,
            ],
            compiler_params=pltpu.CompilerParams(use_tc_tiling_on_sc=False,
                                                 needs_layout_passes=False),
        )
        _sc_built["agg"] = fn
    return fn(g, src, dst, z2)


def _sc_gather_body(t_hbm, e0_hbm, e1_hbm, out0, out1,
                    e0_v, e1_v, a0, a1, b0, b1, sa0, sa1, sb0, sb1):
    c = lax.axis_index("c")
    s = lax.axis_index("s")
    base = jnp.where(c == 0, s * (NCH0 * BC),
                     16 * NCH0 * BC + s * (NCH1 * BC))
    npairs = jnp.where(c == 0, NCH0 // 2, NCH1 // 2)

    @pl.when(c == 0)
    def _():
        pltpu.sync_copy(e0_hbm.at[pl.ds(base, NCH0 * BC)],
                        e0_v.at[pl.ds(0, NCH0 * BC)])
        pltpu.sync_copy(e1_hbm.at[pl.ds(base, NCH0 * BC)],
                        e1_v.at[pl.ds(0, NCH0 * BC)])

    @pl.when(c == 1)
    def _():
        pltpu.sync_copy(e0_hbm.at[pl.ds(base, NCH1 * BC)],
                        e0_v.at[pl.ds(0, NCH1 * BC)])
        pltpu.sync_copy(e1_hbm.at[pl.ds(base, NCH1 * BC)],
                        e1_v.at[pl.ds(0, NCH1 * BC)])

    def start(j, d0, d1, s0, s1):
        o = j * BC
        pltpu.async_copy(t_hbm.at[e0_v.at[pl.ds(o, BC)]], d0, s0)
        pltpu.async_copy(t_hbm.at[e1_v.at[pl.ds(o, BC)]], d1, s1)

    def wait(j, d0, d1, s0, s1):
        o = j * BC
        pltpu.make_async_copy(t_hbm.at[e0_v.at[pl.ds(o, BC)]], d0, s0).wait()
        pltpu.make_async_copy(t_hbm.at[e1_v.at[pl.ds(o, BC)]], d1, s1).wait()

    def copyout(j, d0, d1):
        o = j * BC
        pltpu.sync_copy(d0, out0.at[pl.ds(base + o, BC)])
        pltpu.sync_copy(d1, out1.at[pl.ds(base + o, BC)])

    start(0, a0, a1, sa0, sa1)

    def body(g, carry):
        c0 = 2 * g
        start(c0 + 1, b0, b1, sb0, sb1)
        wait(c0, a0, a1, sa0, sa1)
        copyout(c0, a0, a1)

        @pl.when(g < npairs - 1)
        def _():
            start(c0 + 2, a0, a1, sa0, sa1)

        wait(c0 + 1, b0, b1, sb0, sb1)
        copyout(c0 + 1, b0, b1)
        return carry

    lax.fori_loop(0, npairs, body, 0)


def _sc_gather(t_tab, e0p, e1p):
    fn = _sc_built.get("gather")
    if fn is None:
        fn = pl.kernel(
            _sc_gather_body,
            out_type=(
                jax.ShapeDtypeStruct((BPAD, H), jnp.uint32),
                jax.ShapeDtypeStruct((BPAD, H), jnp.uint32),
            ),
            mesh=_sc_mesh(),
            scratch_types=[
                pltpu.VMEM((NCH0 * BC,), jnp.int32),
                pltpu.VMEM((NCH0 * BC,), jnp.int32),
                pltpu.VMEM((BC, H), jnp.uint32),
                pltpu.VMEM((BC, H), jnp.uint32),
                pltpu.VMEM((BC, H), jnp.uint32),
                pltpu.VMEM((BC, H), jnp.uint32),
                pltpu.SemaphoreType.DMA,
                pltpu.SemaphoreType.DMA,
                pltpu.SemaphoreType.DMA,
                pltpu.SemaphoreType.DMA,
            ],
        )
        _sc_built["gather"] = fn
    return fn(t_tab, e0p, e1p)


def _sc_sp_body(s_hbm, p_hbm, ix_hbm, outs, outp, tab_v, ix_v, ov):
    c = lax.axis_index("c")
    s = lax.axis_index("s")
    wid = c * NS + s
    base = wid * B_PER_TILE
    pltpu.sync_copy(ix_hbm.at[pl.ds(base, B_PER_TILE)], ix_v)

    def gbody(i, carry):
        idx = ix_v[pl.ds(i * 16, 16)]
        ov[pl.ds(i * 16, 16)] = plsc.load_gather(tab_v, [idx])
        return carry

    pltpu.sync_copy(s_hbm, tab_v)
    lax.fori_loop(0, B_PER_TILE // 16, gbody, 0)
    pltpu.sync_copy(ov, outs.at[pl.ds(base, B_PER_TILE)])
    pltpu.sync_copy(p_hbm, tab_v)
    lax.fori_loop(0, B_PER_TILE // 16, gbody, 0)
    pltpu.sync_copy(ov, outp.at[pl.ds(base, B_PER_TILE)])


def _sc_sp(train_s, train_p, ixp):
    fn = _sc_built.get("sp")
    if fn is None:
        fn = pl.kernel(
            _sc_sp_body,
            out_type=(
                jax.ShapeDtypeStruct((BPAD,), jnp.float32),
                jax.ShapeDtypeStruct((BPAD,), jnp.float32),
            ),
            mesh=_sc_mesh(),
            scratch_types=[
                pltpu.VMEM((B,), jnp.float32),
                pltpu.VMEM((B_PER_TILE,), jnp.int32),
                pltpu.VMEM((B_PER_TILE,), jnp.float32),
            ],
            compiler_params=pltpu.CompilerParams(use_tc_tiling_on_sc=False, needs_layout_passes=False),
        )
        _sc_built["sp"] = fn
    return fn(train_s, train_p, ixp)


_R = 1000


def _tca_body(x_ref, w_ref, degs_ref, g_ref, dinv_ref):
    deg = jnp.sum(degs_ref[...], axis=1, keepdims=True) + 1.0
    dinv = lax.rsqrt(jnp.maximum(deg, 1.0))
    g_ref[...] = jnp.dot(x_ref[...], w_ref[...],
                         preferred_element_type=jnp.float32) * dinv
    dinv_ref[...] = dinv


_tc_a = pl.pallas_call(
    _tca_body,
    grid=(N // _R,),
    in_specs=[
        pl.BlockSpec((_R, H), lambda i: (i, 0)),
        pl.BlockSpec((H, H), lambda i: (0, 0)),
        pl.BlockSpec((_R, NW), lambda i: (i, 0)),
    ],
    out_specs=[
        pl.BlockSpec((_R, H), lambda i: (i, 0)),
        pl.BlockSpec((_R, 1), lambda i: (i, 0)),
    ],
    out_shape=[
        jax.ShapeDtypeStruct((N, H), jnp.float32),
        jax.ShapeDtypeStruct((N, 1), jnp.float32),
    ],
)


def _tcb_body(p0_ref, p1_ref, g1_ref, dinv_ref, b1_ref, w2_ref, g2_ref):
    dinv = dinv_ref[...]
    h1 = jnp.tanh(dinv * (p0_ref[...] + p1_ref[...] + g1_ref[...]) + b1_ref[...])
    g2_ref[...] = jnp.dot(h1, w2_ref[...],
                          preferred_element_type=jnp.float32) * dinv


_tc_b = pl.pallas_call(
    _tcb_body,
    grid=(N // _R,),
    in_specs=[
        pl.BlockSpec((_R, H), lambda i: (i, 0)),
        pl.BlockSpec((_R, H), lambda i: (i, 0)),
        pl.BlockSpec((_R, H), lambda i: (i, 0)),
        pl.BlockSpec((_R, 1), lambda i: (i, 0)),
        pl.BlockSpec((1, H), lambda i: (0, 0)),
        pl.BlockSpec((H, H), lambda i: (0, 0)),
    ],
    out_specs=pl.BlockSpec((_R, H), lambda i: (i, 0)),
    out_shape=jax.ShapeDtypeStruct((N, H), jnp.float32),
)


def _tcc_body(p0_ref, p1_ref, g2_ref, dinv_ref, b2_ref, wv_ref, linw_ref,
              linb_ref, msl_ref, redw_ref, redb_ref, wlin_ref,
              t_ref, sym_ref):
    emb0 = jnp.tanh(dinv_ref[...] * (p0_ref[...] + p1_ref[...] + g2_ref[...])
                    + b2_ref[...])
    attn = jnp.dot(emb0, wv_ref[...], preferred_element_type=jnp.float32)
    emb = jnp.tanh(
        jnp.dot(emb0, linw_ref[0:H, :], preferred_element_type=jnp.float32)
        + jnp.dot(attn, linw_ref[H:2 * H, :], preferred_element_type=jnp.float32)
        + linb_ref[...]
    )
    red = jnp.tanh(
        jnp.dot(msl_ref[...], redw_ref[...], preferred_element_type=jnp.float32)
        + redb_ref[...]
    )
    emb_u = lax.bitcast_convert_type(
        emb.astype(jnp.bfloat16).astype(jnp.float32), jnp.uint32)
    red_u = lax.bitcast_convert_type(
        red.astype(jnp.bfloat16).astype(jnp.float32), jnp.uint32)
    t_ref[...] = (emb_u >> 16) | (red_u & jnp.uint32(0xFFFF0000))

    @pl.when(pl.program_id(0) == 0)
    def _():
        w = wlin_ref[...]
        sym_ref[...] = (w + w.T) * 0.5


_tc_c = pl.pallas_call(
    _tcc_body,
    grid=(N // _R,),
    in_specs=[
        pl.BlockSpec((_R, H), lambda i: (i, 0)),
        pl.BlockSpec((_R, H), lambda i: (i, 0)),
        pl.BlockSpec((_R, H), lambda i: (i, 0)),
        pl.BlockSpec((_R, 1), lambda i: (i, 0)),
        pl.BlockSpec((1, H), lambda i: (0, 0)),
        pl.BlockSpec((H, H), lambda i: (0, 0)),
        pl.BlockSpec((2 * H, H), lambda i: (0, 0)),
        pl.BlockSpec((1, H), lambda i: (0, 0)),
        pl.BlockSpec((_R, 64), lambda i: (i, 0)),
        pl.BlockSpec((64, H), lambda i: (0, 0)),
        pl.BlockSpec((1, H), lambda i: (0, 0)),
        pl.BlockSpec((H, H), lambda i: (0, 0)),
    ],
    out_specs=[
        pl.BlockSpec((_R, H), lambda i: (i, 0)),
        pl.BlockSpec((H, H), lambda i: (0, 0)),
    ],
    out_shape=[
        jax.ShapeDtypeStruct((N, H), jnp.uint32),
        jax.ShapeDtypeStruct((H, H), jnp.float32),
    ],
)

_RB = 512


def _tce_body(g0_ref, g1_ref, s_ref, p_ref, sym_ref, mlw1_ref, mlb1_ref,
              mlw2_ref, msw1_ref, msb1_ref, msw2_ref, blin_ref, sc2_ref,
              out_ref):
    u0 = g0_ref[...]
    u1 = g1_ref[...]
    a = lax.bitcast_convert_type(u0 << 16, jnp.float32)
    ra = lax.bitcast_convert_type(u0 & jnp.uint32(0xFFFF0000), jnp.float32)
    b = lax.bitcast_convert_type(u1 << 16, jnp.float32)
    rb = lax.bitcast_convert_type(u1 & jnp.uint32(0xFFFF0000), jnp.float32)

    def bdot(x, w):
        return jnp.dot(x.astype(jnp.bfloat16), w.astype(jnp.bfloat16),
                       preferred_element_type=jnp.float32)

    asym = bdot(a, sym_ref[...])
    sim = jnp.sum(asym * b, axis=1, keepdims=True) + jnp.sum(blin_ref[...])
    ml_s = jax.nn.sigmoid(sim)
    mlh = jnp.tanh(
        bdot((a + b) * 0.5, mlw1_ref[0:H, :])
        + bdot(jnp.maximum(a, b), mlw1_ref[H:2 * H, :])
        + mlb1_ref[...]
    )
    ml_w = jnp.tanh(jnp.sum(mlh * mlw2_ref[...], axis=1, keepdims=True)
                    + sc2_ref[0:1, 0:1])
    msh = jnp.tanh(
        bdot((ra + rb) * 0.5, msw1_ref[0:H, :])
        + bdot(jnp.maximum(ra, rb), msw1_ref[H:2 * H, :])
        + msb1_ref[...]
    )
    ms_w = jnp.tanh(jnp.sum(msh * msw2_ref[...], axis=1, keepdims=True)
                    + sc2_ref[0:1, 1:2])
    m = jnp.maximum(jnp.maximum(ml_w, ms_w), PROX_W)
    e_ml = jnp.exp(ml_w - m)
    e_ms = jnp.exp(ms_w - m)
    e_pw = jnp.exp(PROX_W - m)
    z = e_ml + e_ms + e_pw
    res = (ml_s * e_ml + s_ref[...] * e_ms + p_ref[...] * e_pw) / z
    out_ref[...] = jnp.clip(res, 0.0, 1.0)[:, 0]


_tc_edge = pl.pallas_call(
    _tce_body,
    grid=(BPAD // _RB,),
    in_specs=[
        pl.BlockSpec((_RB, H), lambda i: (i, 0)),
        pl.BlockSpec((_RB, H), lambda i: (i, 0)),
        pl.BlockSpec((_RB, 1), lambda i: (i, 0)),
        pl.BlockSpec((_RB, 1), lambda i: (i, 0)),
        pl.BlockSpec((H, H), lambda i: (0, 0)),
        pl.BlockSpec((2 * H, H), lambda i: (0, 0)),
        pl.BlockSpec((1, H), lambda i: (0, 0)),
        pl.BlockSpec((1, H), lambda i: (0, 0)),
        pl.BlockSpec((2 * H, H), lambda i: (0, 0)),
        pl.BlockSpec((1, H), lambda i: (0, 0)),
        pl.BlockSpec((1, H), lambda i: (0, 0)),
        pl.BlockSpec((1, H), lambda i: (0, 0)),
        pl.BlockSpec((1, 2), lambda i: (0, 0)),
    ],
    out_specs=pl.BlockSpec((_RB,), lambda i: (i,)),
    out_shape=jax.ShapeDtypeStruct((BPAD,), jnp.float32),
)


def kernel(x, mp_adj, edges, index, prev_embs, gc1_W, gc1_b, gc2_W, gc2_b,
           lin_W, lin_b, weight_lin, bias_lin, w_v, train_s, train_p,
           ms_logits, ml_W1, ml_b1, ml_W2, ml_b2, ms_W1, ms_b1, ms_W2, ms_b2,
           red_W, red_b):
    src = mp_adj[0].astype(jnp.int32).reshape(NW, NCH_E, EC)
    dst = mp_adj[1].astype(jnp.int32).reshape(NW, NCH_E, EC)
    dst_flat = mp_adj[1].astype(jnp.int32)
    z2 = jnp.zeros((NP_TILE, H), jnp.float32)

    degp = _sc_deg(dst_flat)
    degs_t = degp.reshape(NW, N_PAD).T[:N]

    g1, dinv = _tc_a(x, gc1_W, degs_t)
    parts1 = _sc_agg(g1, src, dst, z2)
    g2 = _tc_b(parts1[0, :N], parts1[1, :N], g1, dinv,
               gc1_b.reshape(1, H), gc2_W)
    parts2 = _sc_agg(g2, src, dst, z2)
    t_tab, sym = _tc_c(parts2[0, :N], parts2[1, :N], g2, dinv,
                       gc2_b.reshape(1, H), w_v, lin_W, lin_b.reshape(1, H),
                       ms_logits, red_W, red_b.reshape(1, H), weight_lin)

    e0p = jnp.pad(edges[0].astype(jnp.int32), (0, BPAD - B))
    e1p = jnp.pad(edges[1].astype(jnp.int32), (0, BPAD - B))
    ixp = jnp.pad(index.astype(jnp.int32), (0, BPAD - B))

    g0, g1e = _sc_gather(t_tab, e0p, e1p)
    sg, pg = _sc_sp(train_s, train_p, ixp)

    sc2 = jnp.stack([ml_b2[0], ms_b2[0]]).reshape(1, 2)
    out = _tc_edge(g0, g1e, sg.reshape(BPAD, 1), pg.reshape(BPAD, 1), sym,
                   ml_W1, ml_b1.reshape(1, H), ml_W2.reshape(1, H), ms_W1,
                   ms_b1.reshape(1, H), ms_W2.reshape(1, H),
                   bias_lin.reshape(1, H), sc2)
    return out[:B]


# speedup vs baseline: 1.0485x; 1.0485x over previous
import functools

import jax
import jax.numpy as jnp
from jax import lax
from jax.experimental import pallas as pl
from jax.experimental.pallas import tpu as pltpu
from jax.experimental.pallas import tpu_sc as plsc

N = 10000
E = 320000
B = 100000
H = 128
PROX_W = 0.3

NC, NS = 2, 16
NW = NC * NS

E_PER_TILE = E // NW
EC = 80
NCH_E = E_PER_TILE // EC

BPAD = 102400
BC = 80
NCH0 = 62
NCH1 = 18
B_PER_TILE = BPAD // NW

N_PAD = 10016
NP_TILE = N_PAD // NS

_sc_built = {}


def _sc_mesh():
    return plsc.VectorSubcoreMesh(
        core_axis_name="c", subcore_axis_name="s", num_cores=NC, num_subcores=NS
    )


def _sc_deg_body(dst_hbm, out_hbm, dst_v, cnt_v):
    c = lax.axis_index("c")
    s = lax.axis_index("s")
    wid = c * NS + s
    pltpu.sync_copy(dst_hbm.at[pl.ds(wid * E_PER_TILE, E_PER_TILE)], dst_v)

    def zbody(i, carry):
        cnt_v[pl.ds(i * 16, 16)] = jnp.zeros((16,), jnp.float32)
        return carry

    lax.fori_loop(0, N_PAD // 16, zbody, 0)
    ones16 = jnp.full((16,), 1.0, jnp.float32)

    def body(i, carry):
        idx = dst_v[pl.ds(i * 16, 16)]
        plsc.addupdate_scatter(cnt_v, [idx], ones16)
        return carry

    lax.fori_loop(0, E_PER_TILE // 16, body, 0)
    pltpu.sync_copy(cnt_v, out_hbm.at[pl.ds(wid * N_PAD, N_PAD)])


def _sc_deg(dst_flat):
    fn = _sc_built.get("deg")
    if fn is None:
        fn = pl.kernel(
            _sc_deg_body,
            out_type=jax.ShapeDtypeStruct((NW * N_PAD,), jnp.float32),
            mesh=_sc_mesh(),
            scratch_types=[
                pltpu.VMEM((E_PER_TILE,), jnp.int32),
                pltpu.VMEM((N_PAD,), jnp.float32),
            ],
            compiler_params=pltpu.CompilerParams(use_tc_tiling_on_sc=False, needs_layout_passes=False),
        )
        _sc_built["deg"] = fn
    return fn(dst_flat)


def _sc_agg_body(g_hbm, src_hbm, dst_hbm, z2_hbm, out_hbm, src_v, dst_v,
                 rows_a, rows_b, rows_c, acc, sem_a, sem_b, sem_c):
    c = lax.axis_index("c")
    s = lax.axis_index("s")
    wid = c * NS + s
    pltpu.sync_copy(src_hbm.at[wid], src_v)
    pltpu.sync_copy(dst_hbm.at[wid], dst_v)
    pltpu.sync_copy(z2_hbm, acc.at[pl.ds(s * NP_TILE, NP_TILE)])
    plsc.subcore_barrier()

    def start(j, buf, sem):
        return pltpu.async_copy(g_hbm.at[src_v.at[j]], buf, sem)

    def wait_sc(j, buf, sem):
        pltpu.make_async_copy(g_hbm.at[src_v.at[j]], buf, sem).wait()
        pltpu.sync_copy(buf, acc.at[dst_v.at[j]], add=True)

    start(0, rows_a, sem_a)
    start(1, rows_b, sem_b)
    start(2, rows_c, sem_c)

    def body(g, carry):
        c0 = 3 * g
        wait_sc(c0, rows_a, sem_a)
        start(c0 + 3, rows_a, sem_a)
        wait_sc(c0 + 1, rows_b, sem_b)
        start(c0 + 4, rows_b, sem_b)
        wait_sc(c0 + 2, rows_c, sem_c)
        start(c0 + 5, rows_c, sem_c)
        return carry

    lax.fori_loop(0, NCH_E // 3 - 1, body, 0)
    t0 = 3 * (NCH_E // 3 - 1)
    wait_sc(t0, rows_a, sem_a)
    start(t0 + 3, rows_a, sem_a)
    wait_sc(t0 + 1, rows_b, sem_b)
    start(t0 + 4, rows_b, sem_b)
    wait_sc(t0 + 2, rows_c, sem_c)
    wait_sc(t0 + 3, rows_a, sem_a)
    wait_sc(t0 + 4, rows_b, sem_b)
    plsc.subcore_barrier()
    pltpu.sync_copy(
        acc.at[pl.ds(s * NP_TILE, NP_TILE)],
        out_hbm.at[c, pl.ds(s * NP_TILE, NP_TILE)],
    )


def _sc_agg(g, src, dst, z2):
    fn = _sc_built.get("agg")
    if fn is None:
        fn = pl.kernel(
            _sc_agg_body,
            out_type=jax.ShapeDtypeStruct((NC, N_PAD, H), jnp.float32),
            mesh=_sc_mesh(),
            scratch_types=[
                pltpu.VMEM((NCH_E, EC), jnp.int32),
                pltpu.VMEM((NCH_E, EC), jnp.int32),
                pltpu.VMEM((EC, H), jnp.float32),
                pltpu.VMEM((EC, H), jnp.float32),
                pltpu.VMEM((EC, H), jnp.float32),
                pltpu.VMEM_SHARED((N_PAD, H), jnp.float32),
                pltpu.SemaphoreType.DMA,
                pltpu.SemaphoreType.DMA,
                pltpu.SemaphoreType.DMA,
            ],
            compiler_params=pltpu.CompilerParams(use_tc_tiling_on_sc=False,
                                                 needs_layout_passes=False),
        )
        _sc_built["agg"] = fn
    return fn(g, src, dst, z2)


def _sc_gather_body(t_hbm, e0_hbm, e1_hbm, out0, out1,
                    e0_v, e1_v, a0, a1, b0, b1, sa0, sa1, sb0, sb1):
    c = lax.axis_index("c")
    s = lax.axis_index("s")
    base = jnp.where(c == 0, s * (NCH0 * BC),
                     16 * NCH0 * BC + s * (NCH1 * BC))
    npairs = jnp.where(c == 0, NCH0 // 2, NCH1 // 2)

    @pl.when(c == 0)
    def _():
        pltpu.sync_copy(e0_hbm.at[pl.ds(base, NCH0 * BC)],
                        e0_v.at[pl.ds(0, NCH0 * BC)])
        pltpu.sync_copy(e1_hbm.at[pl.ds(base, NCH0 * BC)],
                        e1_v.at[pl.ds(0, NCH0 * BC)])

    @pl.when(c == 1)
    def _():
        pltpu.sync_copy(e0_hbm.at[pl.ds(base, NCH1 * BC)],
                        e0_v.at[pl.ds(0, NCH1 * BC)])
        pltpu.sync_copy(e1_hbm.at[pl.ds(base, NCH1 * BC)],
                        e1_v.at[pl.ds(0, NCH1 * BC)])

    def start(j, d0, d1, s0, s1):
        o = j * BC
        pltpu.async_copy(t_hbm.at[e0_v.at[pl.ds(o, BC)]], d0, s0)
        pltpu.async_copy(t_hbm.at[e1_v.at[pl.ds(o, BC)]], d1, s1)

    def wait(j, d0, d1, s0, s1):
        o = j * BC
        pltpu.make_async_copy(t_hbm.at[e0_v.at[pl.ds(o, BC)]], d0, s0).wait()
        pltpu.make_async_copy(t_hbm.at[e1_v.at[pl.ds(o, BC)]], d1, s1).wait()

    def copyout(j, d0, d1):
        o = j * BC
        pltpu.sync_copy(d0, out0.at[pl.ds(base + o, BC)])
        pltpu.sync_copy(d1, out1.at[pl.ds(base + o, BC)])

    start(0, a0, a1, sa0, sa1)

    def body(g, carry):
        c0 = 2 * g
        start(c0 + 1, b0, b1, sb0, sb1)
        wait(c0, a0, a1, sa0, sa1)
        copyout(c0, a0, a1)

        @pl.when(g < npairs - 1)
        def _():
            start(c0 + 2, a0, a1, sa0, sa1)

        wait(c0 + 1, b0, b1, sb0, sb1)
        copyout(c0 + 1, b0, b1)
        return carry

    lax.fori_loop(0, npairs, body, 0)


def _sc_gather(t_tab, e0p, e1p):
    fn = _sc_built.get("gather")
    if fn is None:
        fn = pl.kernel(
            _sc_gather_body,
            out_type=(
                jax.ShapeDtypeStruct((BPAD, H), jnp.uint32),
                jax.ShapeDtypeStruct((BPAD, H), jnp.uint32),
            ),
            mesh=_sc_mesh(),
            scratch_types=[
                pltpu.VMEM((NCH0 * BC,), jnp.int32),
                pltpu.VMEM((NCH0 * BC,), jnp.int32),
                pltpu.VMEM((BC, H), jnp.uint32),
                pltpu.VMEM((BC, H), jnp.uint32),
                pltpu.VMEM((BC, H), jnp.uint32),
                pltpu.VMEM((BC, H), jnp.uint32),
                pltpu.SemaphoreType.DMA,
                pltpu.SemaphoreType.DMA,
                pltpu.SemaphoreType.DMA,
                pltpu.SemaphoreType.DMA,
            ],
        )
        _sc_built["gather"] = fn
    return fn(t_tab, e0p, e1p)


def _sc_sp_body(s_hbm, p_hbm, ix_hbm, outs, outp, tab_v, ix_v, ov):
    c = lax.axis_index("c")
    s = lax.axis_index("s")
    wid = c * NS + s
    base = wid * B_PER_TILE
    pltpu.sync_copy(ix_hbm.at[pl.ds(base, B_PER_TILE)], ix_v)

    def gbody(i, carry):
        idx = ix_v[pl.ds(i * 16, 16)]
        ov[pl.ds(i * 16, 16)] = plsc.load_gather(tab_v, [idx])
        return carry

    pltpu.sync_copy(s_hbm, tab_v)
    lax.fori_loop(0, B_PER_TILE // 16, gbody, 0)
    pltpu.sync_copy(ov, outs.at[pl.ds(base, B_PER_TILE)])
    pltpu.sync_copy(p_hbm, tab_v)
    lax.fori_loop(0, B_PER_TILE // 16, gbody, 0)
    pltpu.sync_copy(ov, outp.at[pl.ds(base, B_PER_TILE)])


def _sc_sp(train_s, train_p, ixp):
    fn = _sc_built.get("sp")
    if fn is None:
        fn = pl.kernel(
            _sc_sp_body,
            out_type=(
                jax.ShapeDtypeStruct((BPAD,), jnp.float32),
                jax.ShapeDtypeStruct((BPAD,), jnp.float32),
            ),
            mesh=_sc_mesh(),
            scratch_types=[
                pltpu.VMEM((B,), jnp.float32),
                pltpu.VMEM((B_PER_TILE,), jnp.int32),
                pltpu.VMEM((B_PER_TILE,), jnp.float32),
            ],
            compiler_params=pltpu.CompilerParams(use_tc_tiling_on_sc=False, needs_layout_passes=False),
        )
        _sc_built["sp"] = fn
    return fn(train_s, train_p, ixp)


_R = 1000


def _tca_body(x_ref, w_ref, degs_ref, g_ref, dinv_ref):
    deg = jnp.sum(degs_ref[...], axis=1, keepdims=True) + 1.0
    dinv = lax.rsqrt(jnp.maximum(deg, 1.0))
    g_ref[...] = jnp.dot(x_ref[...], w_ref[...],
                         preferred_element_type=jnp.float32) * dinv
    dinv_ref[...] = dinv


_tc_a = pl.pallas_call(
    _tca_body,
    grid=(N // _R,),
    in_specs=[
        pl.BlockSpec((_R, H), lambda i: (i, 0)),
        pl.BlockSpec((H, H), lambda i: (0, 0)),
        pl.BlockSpec((_R, NW), lambda i: (i, 0)),
    ],
    out_specs=[
        pl.BlockSpec((_R, H), lambda i: (i, 0)),
        pl.BlockSpec((_R, 1), lambda i: (i, 0)),
    ],
    out_shape=[
        jax.ShapeDtypeStruct((N, H), jnp.float32),
        jax.ShapeDtypeStruct((N, 1), jnp.float32),
    ],
)


def _tcb_body(p0_ref, p1_ref, g1_ref, dinv_ref, b1_ref, w2_ref, g2_ref):
    dinv = dinv_ref[...]
    h1 = jnp.tanh(dinv * (p0_ref[...] + p1_ref[...] + g1_ref[...]) + b1_ref[...])
    g2_ref[...] = jnp.dot(h1, w2_ref[...],
                          preferred_element_type=jnp.float32) * dinv


_tc_b = pl.pallas_call(
    _tcb_body,
    grid=(N // _R,),
    in_specs=[
        pl.BlockSpec((_R, H), lambda i: (i, 0)),
        pl.BlockSpec((_R, H), lambda i: (i, 0)),
        pl.BlockSpec((_R, H), lambda i: (i, 0)),
        pl.BlockSpec((_R, 1), lambda i: (i, 0)),
        pl.BlockSpec((1, H), lambda i: (0, 0)),
        pl.BlockSpec((H, H), lambda i: (0, 0)),
    ],
    out_specs=pl.BlockSpec((_R, H), lambda i: (i, 0)),
    out_shape=jax.ShapeDtypeStruct((N, H), jnp.float32),
)


def _tcc_body(p0_ref, p1_ref, g2_ref, dinv_ref, b2_ref, wv_ref, linw_ref,
              linb_ref, msl_ref, redw_ref, redb_ref, wlin_ref,
              t_ref, sym_ref):
    emb0 = jnp.tanh(dinv_ref[...] * (p0_ref[...] + p1_ref[...] + g2_ref[...])
                    + b2_ref[...])
    attn = jnp.dot(emb0, wv_ref[...], preferred_element_type=jnp.float32)
    emb = jnp.tanh(
        jnp.dot(emb0, linw_ref[0:H, :], preferred_element_type=jnp.float32)
        + jnp.dot(attn, linw_ref[H:2 * H, :], preferred_element_type=jnp.float32)
        + linb_ref[...]
    )
    red = jnp.tanh(
        jnp.dot(msl_ref[...], redw_ref[...], preferred_element_type=jnp.float32)
        + redb_ref[...]
    )
    emb_u = lax.bitcast_convert_type(
        emb.astype(jnp.bfloat16).astype(jnp.float32), jnp.uint32)
    red_u = lax.bitcast_convert_type(
        red.astype(jnp.bfloat16).astype(jnp.float32), jnp.uint32)
    t_ref[...] = (emb_u >> 16) | (red_u & jnp.uint32(0xFFFF0000))

    @pl.when(pl.program_id(0) == 0)
    def _():
        w = wlin_ref[...]
        sym_ref[...] = (w + w.T) * 0.5


_tc_c = pl.pallas_call(
    _tcc_body,
    grid=(N // _R,),
    in_specs=[
        pl.BlockSpec((_R, H), lambda i: (i, 0)),
        pl.BlockSpec((_R, H), lambda i: (i, 0)),
        pl.BlockSpec((_R, H), lambda i: (i, 0)),
        pl.BlockSpec((_R, 1), lambda i: (i, 0)),
        pl.BlockSpec((1, H), lambda i: (0, 0)),
        pl.BlockSpec((H, H), lambda i: (0, 0)),
        pl.BlockSpec((2 * H, H), lambda i: (0, 0)),
        pl.BlockSpec((1, H), lambda i: (0, 0)),
        pl.BlockSpec((_R, 64), lambda i: (i, 0)),
        pl.BlockSpec((64, H), lambda i: (0, 0)),
        pl.BlockSpec((1, H), lambda i: (0, 0)),
        pl.BlockSpec((H, H), lambda i: (0, 0)),
    ],
    out_specs=[
        pl.BlockSpec((_R, H), lambda i: (i, 0)),
        pl.BlockSpec((H, H), lambda i: (0, 0)),
    ],
    out_shape=[
        jax.ShapeDtypeStruct((N, H), jnp.uint32),
        jax.ShapeDtypeStruct((H, H), jnp.float32),
    ],
)

_RB = 512


def _tce_body(g0_ref, g1_ref, s_ref, p_ref, sym_ref, mlw1_ref, mlb1_ref,
              mlw2_ref, msw1_ref, msb1_ref, msw2_ref, blin_ref, sc2_ref,
              out_ref):
    u0 = g0_ref[...]
    u1 = g1_ref[...]
    a = lax.bitcast_convert_type(u0 << 16, jnp.float32)
    ra = lax.bitcast_convert_type(u0 & jnp.uint32(0xFFFF0000), jnp.float32)
    b = lax.bitcast_convert_type(u1 << 16, jnp.float32)
    rb = lax.bitcast_convert_type(u1 & jnp.uint32(0xFFFF0000), jnp.float32)

    def bdot(x, w):
        return jnp.dot(x.astype(jnp.bfloat16), w.astype(jnp.bfloat16),
                       preferred_element_type=jnp.float32)

    asym = bdot(a, sym_ref[...])
    sim = jnp.sum(asym * b, axis=1, keepdims=True) + jnp.sum(blin_ref[...])
    ml_s = jax.nn.sigmoid(sim)
    mlh = jnp.tanh(
        bdot((a + b) * 0.5, mlw1_ref[0:H, :])
        + bdot(jnp.maximum(a, b), mlw1_ref[H:2 * H, :])
        + mlb1_ref[...]
    )
    ml_w = jnp.tanh(jnp.sum(mlh * mlw2_ref[...], axis=1, keepdims=True)
                    + sc2_ref[0:1, 0:1])
    msh = jnp.tanh(
        bdot((ra + rb) * 0.5, msw1_ref[0:H, :])
        + bdot(jnp.maximum(ra, rb), msw1_ref[H:2 * H, :])
        + msb1_ref[...]
    )
    ms_w = jnp.tanh(jnp.sum(msh * msw2_ref[...], axis=1, keepdims=True)
                    + sc2_ref[0:1, 1:2])
    m = jnp.maximum(jnp.maximum(ml_w, ms_w), PROX_W)
    e_ml = jnp.exp(ml_w - m)
    e_ms = jnp.exp(ms_w - m)
    e_pw = jnp.exp(PROX_W - m)
    z = e_ml + e_ms + e_pw
    res = (ml_s * e_ml + s_ref[...] * e_ms + p_ref[...] * e_pw) / z
    out_ref[...] = jnp.clip(res, 0.0, 1.0)[:, 0]


_tc_edge = pl.pallas_call(
    _tce_body,
    grid=(BPAD // _RB,),
    in_specs=[
        pl.BlockSpec((_RB, H), lambda i: (i, 0)),
        pl.BlockSpec((_RB, H), lambda i: (i, 0)),
        pl.BlockSpec((_RB, 1), lambda i: (i, 0)),
        pl.BlockSpec((_RB, 1), lambda i: (i, 0)),
        pl.BlockSpec((H, H), lambda i: (0, 0)),
        pl.BlockSpec((2 * H, H), lambda i: (0, 0)),
        pl.BlockSpec((1, H), lambda i: (0, 0)),
        pl.BlockSpec((1, H), lambda i: (0, 0)),
        pl.BlockSpec((2 * H, H), lambda i: (0, 0)),
        pl.BlockSpec((1, H), lambda i: (0, 0)),
        pl.BlockSpec((1, H), lambda i: (0, 0)),
        pl.BlockSpec((1, H), lambda i: (0, 0)),
        pl.BlockSpec((1, 2), lambda i: (0, 0)),
    ],
    out_specs=pl.BlockSpec((_RB,), lambda i: (i,)),
    out_shape=jax.ShapeDtypeStruct((BPAD,), jnp.float32),
)


def kernel(x, mp_adj, edges, index, prev_embs, gc1_W, gc1_b, gc2_W, gc2_b,
           lin_W, lin_b, weight_lin, bias_lin, w_v, train_s, train_p,
           ms_logits, ml_W1, ml_b1, ml_W2, ml_b2, ms_W1, ms_b1, ms_W2, ms_b2,
           red_W, red_b):
    src = mp_adj[0].astype(jnp.int32).reshape(NW, NCH_E, EC)
    dst = mp_adj[1].astype(jnp.int32).reshape(NW, NCH_E, EC)
    dst_flat = mp_adj[1].astype(jnp.int32)
    z2 = jnp.zeros((NP_TILE, H), jnp.float32)

    degp = _sc_deg(dst_flat)
    degs_t = degp.reshape(NW, N_PAD).T[:N]

    g1, dinv = _tc_a(x, gc1_W, degs_t)
    parts1 = _sc_agg(g1, src, dst, z2)
    g2 = _tc_b(parts1[0, :N], parts1[1, :N], g1, dinv,
               gc1_b.reshape(1, H), gc2_W)
    parts2 = _sc_agg(g2, src, dst, z2)
    t_tab, sym = _tc_c(parts2[0, :N], parts2[1, :N], g2, dinv,
                       gc2_b.reshape(1, H), w_v, lin_W, lin_b.reshape(1, H),
                       ms_logits, red_W, red_b.reshape(1, H), weight_lin)

    e0p = jnp.pad(edges[0].astype(jnp.int32), (0, BPAD - B))
    e1p = jnp.pad(edges[1].astype(jnp.int32), (0, BPAD - B))
    ixp = jnp.pad(index.astype(jnp.int32), (0, BPAD - B))

    g0, g1e = _sc_gather(t_tab, e0p, e1p)
    sg, pg = _sc_sp(train_s, train_p, ixp)

    sc2 = jnp.stack([ml_b2[0], ms_b2[0]]).reshape(1, 2)
    out = _tc_edge(g0, g1e, sg.reshape(BPAD, 1), pg.reshape(BPAD, 1), sym,
                   ml_W1, ml_b1.reshape(1, H), ml_W2.reshape(1, H), ms_W1,
                   ms_b1.reshape(1, H), ms_W2.reshape(1, H),
                   bias_lin.reshape(1, H), sc2)
    return out[:B]


# speedup vs baseline: 1.1038x; 1.0528x over previous
import functools

import jax
import jax.numpy as jnp
from jax import lax
from jax.experimental import pallas as pl
from jax.experimental.pallas import tpu as pltpu
from jax.experimental.pallas import tpu_sc as plsc

N = 10000
E = 320000
B = 100000
H = 128
PROX_W = 0.3

NC, NS = 2, 16
NW = NC * NS

E_PER_TILE = E // NW
EC = 80
NCH_E = E_PER_TILE // EC

BPAD = 102400
BC = 80
NCH0 = 80
B_PER_TILE = BPAD // NW

N_PAD = 10016
NP_TILE = N_PAD // NS

_sc_built = {}


def _sc_mesh():
    return plsc.VectorSubcoreMesh(
        core_axis_name="c", subcore_axis_name="s", num_cores=NC, num_subcores=NS
    )


def _sc_deg_body(dst_hbm, out_hbm, dst_v, cnt_v):
    c = lax.axis_index("c")
    s = lax.axis_index("s")
    wid = c * NS + s
    pltpu.sync_copy(dst_hbm.at[pl.ds(wid * E_PER_TILE, E_PER_TILE)], dst_v)

    def zbody(i, carry):
        cnt_v[pl.ds(i * 16, 16)] = jnp.zeros((16,), jnp.float32)
        return carry

    lax.fori_loop(0, N_PAD // 16, zbody, 0)
    ones16 = jnp.full((16,), 1.0, jnp.float32)

    def body(i, carry):
        idx = dst_v[pl.ds(i * 16, 16)]
        plsc.addupdate_scatter(cnt_v, [idx], ones16)
        return carry

    lax.fori_loop(0, E_PER_TILE // 16, body, 0)
    pltpu.sync_copy(cnt_v, out_hbm.at[pl.ds(wid * N_PAD, N_PAD)])


def _sc_deg(dst_flat):
    fn = _sc_built.get("deg")
    if fn is None:
        fn = pl.kernel(
            _sc_deg_body,
            out_type=jax.ShapeDtypeStruct((NW * N_PAD,), jnp.float32),
            mesh=_sc_mesh(),
            scratch_types=[
                pltpu.VMEM((E_PER_TILE,), jnp.int32),
                pltpu.VMEM((N_PAD,), jnp.float32),
            ],
            compiler_params=pltpu.CompilerParams(use_tc_tiling_on_sc=False, needs_layout_passes=False),
        )
        _sc_built["deg"] = fn
    return fn(dst_flat)


def _sc_agg_body(g_hbm, src_hbm, dst_hbm, z2_hbm, out_hbm, src_v, dst_v,
                 rows_a, rows_b, rows_c, acc, sem_a, sem_b, sem_c):
    c = lax.axis_index("c")
    s = lax.axis_index("s")
    wid = c * NS + s
    pltpu.sync_copy(src_hbm.at[wid], src_v)
    pltpu.sync_copy(dst_hbm.at[wid], dst_v)
    pltpu.sync_copy(z2_hbm, acc.at[pl.ds(s * NP_TILE, NP_TILE)])
    plsc.subcore_barrier()

    def start(j, buf, sem):
        return pltpu.async_copy(g_hbm.at[src_v.at[j]], buf, sem)

    def wait_sc(j, buf, sem):
        pltpu.make_async_copy(g_hbm.at[src_v.at[j]], buf, sem).wait()
        pltpu.sync_copy(buf, acc.at[dst_v.at[j]], add=True)

    start(0, rows_a, sem_a)
    start(1, rows_b, sem_b)
    start(2, rows_c, sem_c)

    def body(g, carry):
        c0 = 3 * g
        wait_sc(c0, rows_a, sem_a)
        start(c0 + 3, rows_a, sem_a)
        wait_sc(c0 + 1, rows_b, sem_b)
        start(c0 + 4, rows_b, sem_b)
        wait_sc(c0 + 2, rows_c, sem_c)
        start(c0 + 5, rows_c, sem_c)
        return carry

    lax.fori_loop(0, NCH_E // 3 - 1, body, 0)
    t0 = 3 * (NCH_E // 3 - 1)
    wait_sc(t0, rows_a, sem_a)
    start(t0 + 3, rows_a, sem_a)
    wait_sc(t0 + 1, rows_b, sem_b)
    start(t0 + 4, rows_b, sem_b)
    wait_sc(t0 + 2, rows_c, sem_c)
    wait_sc(t0 + 3, rows_a, sem_a)
    wait_sc(t0 + 4, rows_b, sem_b)
    plsc.subcore_barrier()
    pltpu.sync_copy(
        acc.at[pl.ds(s * NP_TILE, NP_TILE)],
        out_hbm.at[c, pl.ds(s * NP_TILE, NP_TILE)],
    )


def _sc_agg(g, src, dst, z2):
    fn = _sc_built.get("agg")
    if fn is None:
        fn = pl.kernel(
            _sc_agg_body,
            out_type=jax.ShapeDtypeStruct((NC, N_PAD, H), jnp.float32),
            mesh=_sc_mesh(),
            scratch_types=[
                pltpu.VMEM((NCH_E, EC), jnp.int32),
                pltpu.VMEM((NCH_E, EC), jnp.int32),
                pltpu.VMEM((EC, H), jnp.float32),
                pltpu.VMEM((EC, H), jnp.float32),
                pltpu.VMEM((EC, H), jnp.float32),
                pltpu.VMEM_SHARED((N_PAD, H), jnp.float32),
                pltpu.SemaphoreType.DMA,
                pltpu.SemaphoreType.DMA,
                pltpu.SemaphoreType.DMA,
            ],
            compiler_params=pltpu.CompilerParams(use_tc_tiling_on_sc=False,
                                                 needs_layout_passes=False),
        )
        _sc_built["agg"] = fn
    return fn(g, src, dst, z2)


def _sc_gather_body(t_hbm, e0_hbm, e1_hbm, out0, out1,
                    e0_v, e1_v, a0, a1, b0, b1, sa0, sa1, sb0, sb1):
    c = lax.axis_index("c")
    s = lax.axis_index("s")
    base = s * (NCH0 * BC)

    def start(j, d0, d1, s0, s1):
        o = j * BC
        pltpu.async_copy(t_hbm.at[e0_v.at[pl.ds(o, BC)]], d0, s0)
        pltpu.async_copy(t_hbm.at[e1_v.at[pl.ds(o, BC)]], d1, s1)

    def wait(j, d0, d1, s0, s1):
        o = j * BC
        pltpu.make_async_copy(t_hbm.at[e0_v.at[pl.ds(o, BC)]], d0, s0).wait()
        pltpu.make_async_copy(t_hbm.at[e1_v.at[pl.ds(o, BC)]], d1, s1).wait()

    def copyout(j, d0, d1):
        o = j * BC
        pltpu.sync_copy(d0, out0.at[pl.ds(base + o, BC)])
        pltpu.sync_copy(d1, out1.at[pl.ds(base + o, BC)])

    @pl.when(c == 0)
    def _():
        pltpu.sync_copy(e0_hbm.at[pl.ds(base, NCH0 * BC)], e0_v)
        pltpu.sync_copy(e1_hbm.at[pl.ds(base, NCH0 * BC)], e1_v)
        start(0, a0, a1, sa0, sa1)

        def body(g, carry):
            c0 = 2 * g
            start(c0 + 1, b0, b1, sb0, sb1)
            wait(c0, a0, a1, sa0, sa1)
            copyout(c0, a0, a1)

            @pl.when(g < NCH0 // 2 - 1)
            def _():
                start(c0 + 2, a0, a1, sa0, sa1)

            wait(c0 + 1, b0, b1, sb0, sb1)
            copyout(c0 + 1, b0, b1)
            return carry

        lax.fori_loop(0, NCH0 // 2, body, 0)


def _sc_gather(t_tab, e0p, e1p):
    fn = _sc_built.get("gather")
    if fn is None:
        fn = pl.kernel(
            _sc_gather_body,
            out_type=(
                jax.ShapeDtypeStruct((BPAD, H), jnp.uint32),
                jax.ShapeDtypeStruct((BPAD, H), jnp.uint32),
            ),
            mesh=_sc_mesh(),
            scratch_types=[
                pltpu.VMEM((NCH0 * BC,), jnp.int32),
                pltpu.VMEM((NCH0 * BC,), jnp.int32),
                pltpu.VMEM((BC, H), jnp.uint32),
                pltpu.VMEM((BC, H), jnp.uint32),
                pltpu.VMEM((BC, H), jnp.uint32),
                pltpu.VMEM((BC, H), jnp.uint32),
                pltpu.SemaphoreType.DMA,
                pltpu.SemaphoreType.DMA,
                pltpu.SemaphoreType.DMA,
                pltpu.SemaphoreType.DMA,
            ],
        )
        _sc_built["gather"] = fn
    return fn(t_tab, e0p, e1p)


def _sc_sp_body(s_hbm, p_hbm, ix_hbm, outs, outp, tab_v, ix_v, ov):
    c = lax.axis_index("c")
    s = lax.axis_index("s")
    wid = c * NS + s
    base = wid * B_PER_TILE
    pltpu.sync_copy(ix_hbm.at[pl.ds(base, B_PER_TILE)], ix_v)

    def gbody(i, carry):
        idx = ix_v[pl.ds(i * 16, 16)]
        ov[pl.ds(i * 16, 16)] = plsc.load_gather(tab_v, [idx])
        return carry

    pltpu.sync_copy(s_hbm, tab_v)
    lax.fori_loop(0, B_PER_TILE // 16, gbody, 0)
    pltpu.sync_copy(ov, outs.at[pl.ds(base, B_PER_TILE)])
    pltpu.sync_copy(p_hbm, tab_v)
    lax.fori_loop(0, B_PER_TILE // 16, gbody, 0)
    pltpu.sync_copy(ov, outp.at[pl.ds(base, B_PER_TILE)])


def _sc_sp(train_s, train_p, ixp):
    fn = _sc_built.get("sp")
    if fn is None:
        fn = pl.kernel(
            _sc_sp_body,
            out_type=(
                jax.ShapeDtypeStruct((BPAD,), jnp.float32),
                jax.ShapeDtypeStruct((BPAD,), jnp.float32),
            ),
            mesh=_sc_mesh(),
            scratch_types=[
                pltpu.VMEM((B,), jnp.float32),
                pltpu.VMEM((B_PER_TILE,), jnp.int32),
                pltpu.VMEM((B_PER_TILE,), jnp.float32),
            ],
            compiler_params=pltpu.CompilerParams(use_tc_tiling_on_sc=False, needs_layout_passes=False),
        )
        _sc_built["sp"] = fn
    return fn(train_s, train_p, ixp)


_R = 1000


def _tca_body(x_ref, w_ref, degs_ref, g_ref, dinv_ref):
    deg = jnp.sum(degs_ref[...], axis=1, keepdims=True) + 1.0
    dinv = lax.rsqrt(jnp.maximum(deg, 1.0))
    g_ref[...] = jnp.dot(x_ref[...], w_ref[...],
                         preferred_element_type=jnp.float32) * dinv
    dinv_ref[...] = dinv


_tc_a = pl.pallas_call(
    _tca_body,
    grid=(N // _R,),
    in_specs=[
        pl.BlockSpec((_R, H), lambda i: (i, 0)),
        pl.BlockSpec((H, H), lambda i: (0, 0)),
        pl.BlockSpec((_R, NW), lambda i: (i, 0)),
    ],
    out_specs=[
        pl.BlockSpec((_R, H), lambda i: (i, 0)),
        pl.BlockSpec((_R, 1), lambda i: (i, 0)),
    ],
    out_shape=[
        jax.ShapeDtypeStruct((N, H), jnp.float32),
        jax.ShapeDtypeStruct((N, 1), jnp.float32),
    ],
)


def _tcb_body(p0_ref, p1_ref, g1_ref, dinv_ref, b1_ref, w2_ref, g2_ref):
    dinv = dinv_ref[...]
    h1 = jnp.tanh(dinv * (p0_ref[...] + p1_ref[...] + g1_ref[...]) + b1_ref[...])
    g2_ref[...] = jnp.dot(h1, w2_ref[...],
                          preferred_element_type=jnp.float32) * dinv


_tc_b = pl.pallas_call(
    _tcb_body,
    grid=(N // _R,),
    in_specs=[
        pl.BlockSpec((_R, H), lambda i: (i, 0)),
        pl.BlockSpec((_R, H), lambda i: (i, 0)),
        pl.BlockSpec((_R, H), lambda i: (i, 0)),
        pl.BlockSpec((_R, 1), lambda i: (i, 0)),
        pl.BlockSpec((1, H), lambda i: (0, 0)),
        pl.BlockSpec((H, H), lambda i: (0, 0)),
    ],
    out_specs=pl.BlockSpec((_R, H), lambda i: (i, 0)),
    out_shape=jax.ShapeDtypeStruct((N, H), jnp.float32),
)


def _tcc_body(p0_ref, p1_ref, g2_ref, dinv_ref, b2_ref, wv_ref, linw_ref,
              linb_ref, msl_ref, redw_ref, redb_ref, wlin_ref,
              t_ref, sym_ref):
    emb0 = jnp.tanh(dinv_ref[...] * (p0_ref[...] + p1_ref[...] + g2_ref[...])
                    + b2_ref[...])
    attn = jnp.dot(emb0, wv_ref[...], preferred_element_type=jnp.float32)
    emb = jnp.tanh(
        jnp.dot(emb0, linw_ref[0:H, :], preferred_element_type=jnp.float32)
        + jnp.dot(attn, linw_ref[H:2 * H, :], preferred_element_type=jnp.float32)
        + linb_ref[...]
    )
    red = jnp.tanh(
        jnp.dot(msl_ref[...], redw_ref[...], preferred_element_type=jnp.float32)
        + redb_ref[...]
    )
    emb_u = lax.bitcast_convert_type(
        emb.astype(jnp.bfloat16).astype(jnp.float32), jnp.uint32)
    red_u = lax.bitcast_convert_type(
        red.astype(jnp.bfloat16).astype(jnp.float32), jnp.uint32)
    t_ref[...] = (emb_u >> 16) | (red_u & jnp.uint32(0xFFFF0000))

    @pl.when(pl.program_id(0) == 0)
    def _():
        w = wlin_ref[...]
        sym_ref[...] = (w + w.T) * 0.5


_tc_c = pl.pallas_call(
    _tcc_body,
    grid=(N // _R,),
    in_specs=[
        pl.BlockSpec((_R, H), lambda i: (i, 0)),
        pl.BlockSpec((_R, H), lambda i: (i, 0)),
        pl.BlockSpec((_R, H), lambda i: (i, 0)),
        pl.BlockSpec((_R, 1), lambda i: (i, 0)),
        pl.BlockSpec((1, H), lambda i: (0, 0)),
        pl.BlockSpec((H, H), lambda i: (0, 0)),
        pl.BlockSpec((2 * H, H), lambda i: (0, 0)),
        pl.BlockSpec((1, H), lambda i: (0, 0)),
        pl.BlockSpec((_R, 64), lambda i: (i, 0)),
        pl.BlockSpec((64, H), lambda i: (0, 0)),
        pl.BlockSpec((1, H), lambda i: (0, 0)),
        pl.BlockSpec((H, H), lambda i: (0, 0)),
    ],
    out_specs=[
        pl.BlockSpec((_R, H), lambda i: (i, 0)),
        pl.BlockSpec((H, H), lambda i: (0, 0)),
    ],
    out_shape=[
        jax.ShapeDtypeStruct((N, H), jnp.uint32),
        jax.ShapeDtypeStruct((H, H), jnp.float32),
    ],
)

_RB = 512


def _tce_body(g0_ref, g1_ref, s_ref, p_ref, sym_ref, mlw1_ref, mlb1_ref,
              mlw2_ref, msw1_ref, msb1_ref, msw2_ref, blin_ref, sc2_ref,
              out_ref):
    u0 = g0_ref[...]
    u1 = g1_ref[...]
    a = lax.bitcast_convert_type(u0 << 16, jnp.float32)
    ra = lax.bitcast_convert_type(u0 & jnp.uint32(0xFFFF0000), jnp.float32)
    b = lax.bitcast_convert_type(u1 << 16, jnp.float32)
    rb = lax.bitcast_convert_type(u1 & jnp.uint32(0xFFFF0000), jnp.float32)

    def dot(x, w):
        return jnp.dot(x, w, preferred_element_type=jnp.float32)

    def rowsum_t(x, w_col):
        return lax.dot_general(w_col, x, (((0,), (1,)), ((), ())),
                               preferred_element_type=jnp.float32)

    ones_col = jnp.ones((H, 1), jnp.float32)
    asym = dot(a, sym_ref[...])
    sim = rowsum_t(asym * b, ones_col) + jnp.sum(blin_ref[...])
    ml_s = jax.nn.sigmoid(sim)
    mlh = jnp.tanh(
        dot((a + b) * 0.5, mlw1_ref[0:H, :])
        + dot(jnp.maximum(a, b), mlw1_ref[H:2 * H, :])
        + mlb1_ref[...]
    )
    ml_w = jnp.tanh(rowsum_t(mlh, mlw2_ref[...]) + sc2_ref[0:1, 0:1])
    msh = jnp.tanh(
        dot((ra + rb) * 0.5, msw1_ref[0:H, :])
        + dot(jnp.maximum(ra, rb), msw1_ref[H:2 * H, :])
        + msb1_ref[...]
    )
    ms_w = jnp.tanh(rowsum_t(msh, msw2_ref[...]) + sc2_ref[0:1, 1:2])
    m = jnp.maximum(jnp.maximum(ml_w, ms_w), PROX_W)
    e_ml = jnp.exp(ml_w - m)
    e_ms = jnp.exp(ms_w - m)
    e_pw = jnp.exp(PROX_W - m)
    z = e_ml + e_ms + e_pw
    res = (ml_s * e_ml + s_ref[0, :, :] * e_ms + p_ref[0, :, :] * e_pw) / z
    out_ref[...] = jnp.clip(res, 0.0, 1.0).reshape(1, 1, _RB)


_tc_edge = pl.pallas_call(
    _tce_body,
    grid=(BPAD // _RB,),
    in_specs=[
        pl.BlockSpec((_RB, H), lambda i: (i, 0)),
        pl.BlockSpec((_RB, H), lambda i: (i, 0)),
        pl.BlockSpec((1, 1, _RB), lambda i: (i, 0, 0)),
        pl.BlockSpec((1, 1, _RB), lambda i: (i, 0, 0)),
        pl.BlockSpec((H, H), lambda i: (0, 0)),
        pl.BlockSpec((2 * H, H), lambda i: (0, 0)),
        pl.BlockSpec((1, H), lambda i: (0, 0)),
        pl.BlockSpec((H, 1), lambda i: (0, 0)),
        pl.BlockSpec((2 * H, H), lambda i: (0, 0)),
        pl.BlockSpec((1, H), lambda i: (0, 0)),
        pl.BlockSpec((H, 1), lambda i: (0, 0)),
        pl.BlockSpec((1, H), lambda i: (0, 0)),
        pl.BlockSpec((1, 2), lambda i: (0, 0)),
    ],
    out_specs=pl.BlockSpec((1, 1, _RB), lambda i: (i, 0, 0)),
    out_shape=jax.ShapeDtypeStruct((BPAD // _RB, 1, _RB), jnp.float32),
)


def kernel(x, mp_adj, edges, index, prev_embs, gc1_W, gc1_b, gc2_W, gc2_b,
           lin_W, lin_b, weight_lin, bias_lin, w_v, train_s, train_p,
           ms_logits, ml_W1, ml_b1, ml_W2, ml_b2, ms_W1, ms_b1, ms_W2, ms_b2,
           red_W, red_b):
    src = mp_adj[0].astype(jnp.int32).reshape(NW, NCH_E, EC)
    dst = mp_adj[1].astype(jnp.int32).reshape(NW, NCH_E, EC)
    dst_flat = mp_adj[1].astype(jnp.int32)
    z2 = jnp.zeros((NP_TILE, H), jnp.float32)

    degp = _sc_deg(dst_flat)
    degs_t = degp.reshape(NW, N_PAD).T[:N]

    g1, dinv = _tc_a(x, gc1_W, degs_t)
    parts1 = _sc_agg(g1, src, dst, z2)
    g2 = _tc_b(parts1[0, :N], parts1[1, :N], g1, dinv,
               gc1_b.reshape(1, H), gc2_W)
    parts2 = _sc_agg(g2, src, dst, z2)
    t_tab, sym = _tc_c(parts2[0, :N], parts2[1, :N], g2, dinv,
                       gc2_b.reshape(1, H), w_v, lin_W, lin_b.reshape(1, H),
                       ms_logits, red_W, red_b.reshape(1, H), weight_lin)

    e0p = jnp.pad(edges[0].astype(jnp.int32), (0, BPAD - B))
    e1p = jnp.pad(edges[1].astype(jnp.int32), (0, BPAD - B))
    ixp = jnp.pad(index.astype(jnp.int32), (0, BPAD - B))

    g0, g1e = _sc_gather(t_tab, e0p, e1p)
    sg, pg = _sc_sp(train_s, train_p, ixp)

    sc2 = jnp.stack([ml_b2[0], ms_b2[0]]).reshape(1, 2)
    out = _tc_edge(g0, g1e, sg.reshape(BPAD // _RB, 1, _RB),
                   pg.reshape(BPAD // _RB, 1, _RB), sym,
                   ml_W1, ml_b1.reshape(1, H), ml_W2, ms_W1,
                   ms_b1.reshape(1, H), ms_W2,
                   bias_lin.reshape(1, H), sc2)
    return out.reshape(BPAD)[:B]
